# baseline (device time: 144640 ns/iter reference)
import jax
import jax.numpy as jnp
from jax import lax
from jax.experimental import pallas as pl
from jax.experimental.pallas import tpu as pltpu

T = 4096
D = 2048
V_LOCAL = 8192
HALF = T // 2
Q = 32
C = 16
S = HALF // C
YLAG = 2


def kernel(ids, E):
    my_x = lax.axis_index("x")
    my_y = lax.axis_index("y")

    ids_half = lax.dynamic_slice(ids, (my_y * HALF,), (HALF,))
    local = ids_half - my_x * V_LOCAL
    ok = (local >= 0) & (local < V_LOCAL)
    lids = jnp.where(ok, local, -1)
    cnt = jnp.sum(ok.reshape(C, S).astype(jnp.int32), axis=1)
    order = jnp.argsort(jnp.where(ok, 0, 1).astype(jnp.int32), stable=True)
    dpos = order.astype(jnp.int32)
    sidx = lids[order]
    cum = jnp.concatenate([jnp.zeros((1,), jnp.int32), jnp.cumsum(cnt)])

    def body(sidx_ref, dpos_ref, cnt_ref, cum_ref, e_ref, out_ref,
             g, xs, cx, ys, cy, stage,
             csem, sx, rx, sy, ry, copy_sem, ssem):
        x = lax.axis_index("x")
        y = lax.axis_index("y")
        x_nbr = (1 - x, y)
        y_nbr = (x, 1 - y)

        barrier_sem = pltpu.get_barrier_semaphore()
        for nbr in (x_nbr, y_nbr):
            pl.semaphore_signal(
                barrier_sem, inc=1,
                device_id=nbr, device_id_type=pl.DeviceIdType.MESH,
            )

        g[:, :] = jnp.zeros_like(g)

        def issue_chunk(c):
            def one(j, k):
                pltpu.make_async_copy(
                    e_ref.at[pl.ds(sidx_ref[j], 1), :],
                    g.at[pl.ds(dpos_ref[j], 1), :],
                    csem.at[c],
                ).start()
                return k

            lax.fori_loop(cum_ref[c], cum_ref[c + 1], one, 0)

        def gather_wait(c):
            def one(_, k):
                pltpu.make_async_copy(
                    e_ref.at[pl.ds(0, 1), :],
                    g.at[pl.ds(0, 1), :],
                    csem.at[c],
                ).wait()
                return k

            lax.fori_loop(0, cnt_ref[c], one, 0)

        ds = lambda c: pl.ds(c * S, S)

        def make_x(c):
            return pltpu.make_async_remote_copy(
                src_ref=xs.at[ds(c)],
                dst_ref=cx.at[ds(c)],
                send_sem=sx.at[c],
                recv_sem=rx.at[c],
                device_id=x_nbr,
                device_id_type=pl.DeviceIdType.MESH,
            )

        def make_y(c):
            return pltpu.make_async_remote_copy(
                src_ref=ys.at[ds(c)],
                dst_ref=cy.at[ds(c)],
                send_sem=sy.at[c],
                recv_sem=ry.at[c],
                device_id=y_nbr,
                device_id_type=pl.DeviceIdType.MESH,
            )

        def make_local(c):
            return pltpu.make_async_copy(
                g.at[ds(c)],
                out_ref.at[pl.ds(y * HALF + c * S, S), :],
                copy_sem.at[c],
            )

        def make_stage(c):
            return pltpu.make_async_copy(
                stage.at[c % 2],
                out_ref.at[pl.ds((1 - y) * HALF + c * S, S), :],
                ssem.at[c % 2],
            )

        def finish_x(c):
            make_x(c).wait_recv()
            g[ds(c), :] = g[ds(c), :] + cx[ds(c), :].astype(jnp.float32)
            make_local(c).start()
            ys[ds(c), :] = g[ds(c), :].astype(jnp.bfloat16)
            make_y(c).start()

        def finish_y(c):
            make_y(c).wait_recv()
            if c >= 2:
                make_stage(c - 2).wait()
            stage[c % 2, :, :] = cy[ds(c), :].astype(jnp.float32)
            make_stage(c).start()

        issue_chunk(0)
        for c in range(C):
            if c + 1 < C:
                issue_chunk(c + 1)
            gather_wait(c)
            xs[ds(c), :] = g[ds(c), :].astype(jnp.bfloat16)
            if c == 0:
                pl.semaphore_wait(barrier_sem, 2)
            make_x(c).start()
            if c >= 1:
                finish_x(c - 1)
            if c >= 1 + YLAG:
                finish_y(c - 1 - YLAG)
        finish_x(C - 1)
        for c in range(C - YLAG - 1, C):
            finish_y(c)

        for c in range(C):
            make_x(c).wait_send()
            make_y(c).wait_send()
            make_local(c).wait()
        make_stage(C - 2).wait()
        make_stage(C - 1).wait()

    return pl.pallas_call(
        body,
        out_shape=jax.ShapeDtypeStruct((T, D), jnp.float32),
        in_specs=[
            pl.BlockSpec(memory_space=pltpu.SMEM),
            pl.BlockSpec(memory_space=pltpu.SMEM),
            pl.BlockSpec(memory_space=pltpu.SMEM),
            pl.BlockSpec(memory_space=pltpu.SMEM),
            pl.BlockSpec(memory_space=pltpu.MemorySpace.HBM),
        ],
        out_specs=pl.BlockSpec(memory_space=pltpu.MemorySpace.HBM),
        scratch_shapes=[
            pltpu.VMEM((HALF, D), jnp.float32),
            pltpu.VMEM((HALF, D), jnp.bfloat16),
            pltpu.VMEM((HALF, D), jnp.bfloat16),
            pltpu.VMEM((HALF, D), jnp.bfloat16),
            pltpu.VMEM((HALF, D), jnp.bfloat16),
            pltpu.VMEM((2, S, D), jnp.float32),
            pltpu.SemaphoreType.DMA((C,)),
            pltpu.SemaphoreType.DMA((C,)),
            pltpu.SemaphoreType.DMA((C,)),
            pltpu.SemaphoreType.DMA((C,)),
            pltpu.SemaphoreType.DMA((C,)),
            pltpu.SemaphoreType.DMA((C,)),
            pltpu.SemaphoreType.DMA((2,)),
        ],
        compiler_params=pltpu.CompilerParams(
            collective_id=0,
            vmem_limit_bytes=100 * 1024 * 1024,
        ),
    )(sidx, dpos, cnt, cum, E)


# device time: 134734 ns/iter; 1.0735x vs baseline; 1.0735x over previous
import jax
import jax.numpy as jnp
from jax import lax
from jax.experimental import pallas as pl
from jax.experimental.pallas import tpu as pltpu

T = 4096
D = 2048
V_LOCAL = 8192
HALF = T // 2
Q = 32
C = 16
S = HALF // C
YLAG = 2


def kernel(ids, E):
    my_x = lax.axis_index("x")
    my_y = lax.axis_index("y")

    ids_half = lax.dynamic_slice(ids, (my_y * HALF,), (HALF,))
    local = ids_half - my_x * V_LOCAL
    ok = (local >= 0) & (local < V_LOCAL)
    lids = jnp.clip(local, 0, V_LOCAL - 1)
    mask = ok.astype(jnp.float32)[:, None]

    def body(lids_ref, mask_ref, e_ref, out_ref,
             g, xs, cx, ys, cy, stage,
             csem, sx, rx, sy, ry, copy_sem, ssem):
        x = lax.axis_index("x")
        y = lax.axis_index("y")
        x_nbr = (1 - x, y)
        y_nbr = (x, 1 - y)

        barrier_sem = pltpu.get_barrier_semaphore()
        for nbr in (x_nbr, y_nbr):
            pl.semaphore_signal(
                barrier_sem, inc=1,
                device_id=nbr, device_id_type=pl.DeviceIdType.MESH,
            )

        def issue_chunk(c):
            lo, hi = c * S, (c + 1) * S

            def one(i, k):
                pltpu.make_async_copy(
                    e_ref.at[pl.ds(lids_ref[i], 1), :],
                    g.at[pl.ds(i, 1), :],
                    csem.at[c],
                ).start()
                return k

            lax.fori_loop(lo, hi, one, 0, unroll=8)

        def gather_wait(c):
            pltpu.make_async_copy(
                e_ref.at[pl.ds(0, S), :],
                g.at[ds(c)],
                csem.at[c],
            ).wait()

        ds = lambda c: pl.ds(c * S, S)

        def make_x(c):
            return pltpu.make_async_remote_copy(
                src_ref=xs.at[ds(c)],
                dst_ref=cx.at[ds(c)],
                send_sem=sx.at[c],
                recv_sem=rx.at[c],
                device_id=x_nbr,
                device_id_type=pl.DeviceIdType.MESH,
            )

        def make_y(c):
            return pltpu.make_async_remote_copy(
                src_ref=ys.at[ds(c)],
                dst_ref=cy.at[ds(c)],
                send_sem=sy.at[c],
                recv_sem=ry.at[c],
                device_id=y_nbr,
                device_id_type=pl.DeviceIdType.MESH,
            )

        def make_local(c):
            return pltpu.make_async_copy(
                g.at[ds(c)],
                out_ref.at[pl.ds(y * HALF + c * S, S), :],
                copy_sem.at[c],
            )

        def make_stage(c):
            return pltpu.make_async_copy(
                stage.at[c % 2],
                out_ref.at[pl.ds((1 - y) * HALF + c * S, S), :],
                ssem.at[c % 2],
            )

        def finish_x(c):
            make_x(c).wait_recv()
            g[ds(c), :] = (g[ds(c), :] * mask_ref[ds(c), :]
                           + cx[ds(c), :].astype(jnp.float32))
            make_local(c).start()
            ys[ds(c), :] = g[ds(c), :].astype(jnp.bfloat16)
            make_y(c).start()

        def finish_y(c):
            make_y(c).wait_recv()
            if c >= 2:
                make_stage(c - 2).wait()
            stage[c % 2, :, :] = cy[ds(c), :].astype(jnp.float32)
            make_stage(c).start()

        issue_chunk(0)
        for c in range(C):
            if c + 1 < C:
                issue_chunk(c + 1)
            gather_wait(c)
            xs[ds(c), :] = (g[ds(c), :]
                            * mask_ref[ds(c), :]).astype(jnp.bfloat16)
            if c == 0:
                pl.semaphore_wait(barrier_sem, 2)
            make_x(c).start()
            if c >= 1:
                finish_x(c - 1)
            if c >= 1 + YLAG:
                finish_y(c - 1 - YLAG)
        finish_x(C - 1)
        for c in range(C - YLAG - 1, C):
            finish_y(c)

        for c in range(C):
            make_x(c).wait_send()
            make_y(c).wait_send()
            make_local(c).wait()
        make_stage(C - 2).wait()
        make_stage(C - 1).wait()

    return pl.pallas_call(
        body,
        out_shape=jax.ShapeDtypeStruct((T, D), jnp.float32),
        in_specs=[
            pl.BlockSpec(memory_space=pltpu.SMEM),
            pl.BlockSpec(memory_space=pltpu.VMEM),
            pl.BlockSpec(memory_space=pltpu.MemorySpace.HBM),
        ],
        out_specs=pl.BlockSpec(memory_space=pltpu.MemorySpace.HBM),
        scratch_shapes=[
            pltpu.VMEM((HALF, D), jnp.float32),
            pltpu.VMEM((HALF, D), jnp.bfloat16),
            pltpu.VMEM((HALF, D), jnp.bfloat16),
            pltpu.VMEM((HALF, D), jnp.bfloat16),
            pltpu.VMEM((HALF, D), jnp.bfloat16),
            pltpu.VMEM((2, S, D), jnp.float32),
            pltpu.SemaphoreType.DMA((C,)),
            pltpu.SemaphoreType.DMA((C,)),
            pltpu.SemaphoreType.DMA((C,)),
            pltpu.SemaphoreType.DMA((C,)),
            pltpu.SemaphoreType.DMA((C,)),
            pltpu.SemaphoreType.DMA((C,)),
            pltpu.SemaphoreType.DMA((2,)),
        ],
        compiler_params=pltpu.CompilerParams(
            collective_id=0,
            vmem_limit_bytes=100 * 1024 * 1024,
        ),
    )(lids, mask, E)


# device time: 134706 ns/iter; 1.0737x vs baseline; 1.0002x over previous
import jax
import jax.numpy as jnp
from jax import lax
from jax.experimental import pallas as pl
from jax.experimental.pallas import tpu as pltpu

T = 4096
D = 2048
V_LOCAL = 8192
HALF = T // 2
C = 16
S = HALF // C
YLAG = 2


def kernel(ids, E):
    my_x = lax.axis_index("x")
    my_y = lax.axis_index("y")

    ids_half = lax.dynamic_slice(ids, (my_y * HALF,), (HALF,))
    local = ids_half - my_x * V_LOCAL
    ok = (local >= 0) & (local < V_LOCAL)
    lids = jnp.clip(local, 0, V_LOCAL - 1)
    mask = ok.astype(jnp.float32)[:, None]

    def body(lids_ref, mask_ref, e_ref, out_ref,
             g, xs, cx, ys, cy, stage,
             csem, sx, rx, sy, ry, copy_sem, ssem):
        x = lax.axis_index("x")
        y = lax.axis_index("y")
        x_nbr = (1 - x, y)
        y_nbr = (x, 1 - y)

        barrier_sem = pltpu.get_barrier_semaphore()
        for nbr in (x_nbr, y_nbr):
            pl.semaphore_signal(
                barrier_sem, inc=1,
                device_id=nbr, device_id_type=pl.DeviceIdType.MESH,
            )

        def issue_chunk(c):
            lo, hi = c * S, (c + 1) * S

            def one(i, k):
                pltpu.make_async_copy(
                    e_ref.at[pl.ds(lids_ref[i], 1), :],
                    g.at[pl.ds(i, 1), :],
                    csem.at[c],
                ).start()
                return k

            lax.fori_loop(lo, hi, one, 0, unroll=8)

        def gather_wait(c):
            pltpu.make_async_copy(
                e_ref.at[pl.ds(0, S), :],
                g.at[ds(c)],
                csem.at[c],
            ).wait()

        ds = lambda c: pl.ds(c * S, S)

        def make_x(c):
            return pltpu.make_async_remote_copy(
                src_ref=xs.at[ds(c)],
                dst_ref=cx.at[ds(c)],
                send_sem=sx.at[c],
                recv_sem=rx.at[c],
                device_id=x_nbr,
                device_id_type=pl.DeviceIdType.MESH,
            )

        def make_y(c):
            return pltpu.make_async_remote_copy(
                src_ref=ys.at[ds(c)],
                dst_ref=cy.at[ds(c)],
                send_sem=sy.at[c],
                recv_sem=ry.at[c],
                device_id=y_nbr,
                device_id_type=pl.DeviceIdType.MESH,
            )

        def make_local(c):
            return pltpu.make_async_copy(
                g.at[ds(c)],
                out_ref.at[pl.ds(y * HALF + c * S, S), :],
                copy_sem.at[c],
            )

        def make_stage(c):
            return pltpu.make_async_copy(
                stage.at[c % 2],
                out_ref.at[pl.ds((1 - y) * HALF + c * S, S), :],
                ssem.at[c % 2],
            )

        def finish_x(c):
            make_x(c).wait_recv()
            g[ds(c), :] = (g[ds(c), :] * mask_ref[ds(c), :]
                           + cx[ds(c), :].astype(jnp.float32))
            make_local(c).start()
            ys[ds(c), :] = g[ds(c), :].astype(jnp.bfloat16)
            make_y(c).start()

        def finish_y(c):
            make_y(c).wait_recv()
            if c >= 2:
                make_stage(c - 2).wait()
            stage[c % 2, :, :] = cy[ds(c), :].astype(jnp.float32)
            make_stage(c).start()

        issue_chunk(0)
        for c in range(C):
            if c + 1 < C:
                issue_chunk(c + 1)
            gather_wait(c)
            xs[ds(c), :] = (g[ds(c), :]
                            * mask_ref[ds(c), :]).astype(jnp.bfloat16)
            if c == 0:
                pl.semaphore_wait(barrier_sem, 2)
            make_x(c).start()
            if c >= 1:
                finish_x(c - 1)
            if c >= 1 + YLAG:
                finish_y(c - 1 - YLAG)
        finish_x(C - 1)
        for c in range(C - YLAG - 1, C):
            finish_y(c)

        for c in range(C):
            make_x(c).wait_send()
            make_y(c).wait_send()
            make_local(c).wait()
        make_stage(C - 2).wait()
        make_stage(C - 1).wait()

    return pl.pallas_call(
        body,
        out_shape=jax.ShapeDtypeStruct((T, D), jnp.float32),
        in_specs=[
            pl.BlockSpec(memory_space=pltpu.SMEM),
            pl.BlockSpec(memory_space=pltpu.VMEM),
            pl.BlockSpec(memory_space=pltpu.MemorySpace.HBM),
        ],
        out_specs=pl.BlockSpec(memory_space=pltpu.MemorySpace.HBM),
        scratch_shapes=[
            pltpu.VMEM((HALF, D), jnp.float32),
            pltpu.VMEM((HALF, D), jnp.bfloat16),
            pltpu.VMEM((HALF, D), jnp.bfloat16),
            pltpu.VMEM((HALF, D), jnp.bfloat16),
            pltpu.VMEM((HALF, D), jnp.bfloat16),
            pltpu.VMEM((2, S, D), jnp.float32),
            pltpu.SemaphoreType.DMA((C,)),
            pltpu.SemaphoreType.DMA((C,)),
            pltpu.SemaphoreType.DMA((C,)),
            pltpu.SemaphoreType.DMA((C,)),
            pltpu.SemaphoreType.DMA((C,)),
            pltpu.SemaphoreType.DMA((C,)),
            pltpu.SemaphoreType.DMA((2,)),
        ],
        compiler_params=pltpu.CompilerParams(
            collective_id=0,
            vmem_limit_bytes=100 * 1024 * 1024,
        ),
    )(lids, mask, E)
